# baseline (speedup 1.0000x reference)
import jax
import jax.numpy as jnp
from jax import lax
from jax.experimental import pallas as pl
from jax.experimental.pallas import tpu as pltpu
from jax.experimental.pallas import tpu_sc as plsc

_K = 26
_NEMBED = 32
_NHID = 128
_B = 16384
_D = _K * _NEMBED

_INFO = plsc.get_sparse_core_info()
_NC = _INFO.num_cores
_NS = _INFO.num_subcores
_NW = _NC * _NS
_CB = 8
_NCH = _B // (_NW * _CB)
_GR = _CB * _K


def _sc_body(cids_hbm, vids_hbm, cnt_hbm, val_hbm, h_hbm,
             idx_c, idx_v, idx_gc, idx_gv, slab_c, slab_v, buf, sem):
    wid = lax.axis_index("s") * _NC + lax.axis_index("c")

    def per_chunk(c, carry):
        i0 = pl.multiple_of((wid * _NCH + c) * _GR, _GR)
        b0 = pl.multiple_of((wid * _NCH + c) * _CB, _CB)
        pltpu.sync_copy(cids_hbm.at[pl.ds(i0, _GR)], idx_c.at[pl.ds(0, _GR)])
        pltpu.sync_copy(vids_hbm.at[pl.ds(i0, _GR)], idx_v.at[pl.ds(0, _GR)])

        def shift_v(t, carry2):
            sl = pl.ds(t * 16, 16)
            idx_gc[sl] = lax.shift_right_logical(idx_c[sl], 2)
            idx_gv[sl] = lax.shift_right_logical(idx_v[sl], 2)
            return carry2

        lax.fori_loop(0, _GR // 16, shift_v, 0, unroll=4)
        cpy_c = pltpu.async_copy(cnt_hbm.at[idx_gc], slab_c, sem)
        cpy_v = pltpu.async_copy(val_hbm.at[idx_gv], slab_v, sem)
        cpy_c.wait()
        cpy_v.wait()

        def mul_b(b, carry2):
            r0 = b * _K
            vc0 = idx_c[pl.ds(r0, 16)]
            vc1 = idx_c[pl.ds(r0 + 16, 16)]
            vv0 = idx_v[pl.ds(r0, 16)]
            vv1 = idx_v[pl.ds(r0 + 16, 16)]
            for k in range(_K):
                rc = vc0[k] if k < 16 else vc1[k - 16]
                rv = vv0[k] if k < 16 else vv1[k - 16]
                off_c = pl.multiple_of((rc & 3) * _NEMBED, _NEMBED)
                off_v = pl.multiple_of((rv & 3) * _NEMBED, _NEMBED)
                r = r0 + k
                for j in (0, 16):
                    a = slab_c[r, pl.ds(pl.multiple_of(off_c + j, 16), 16)]
                    bb = slab_v[r, pl.ds(pl.multiple_of(off_v + j, 16), 16)]
                    buf[b, pl.ds(k * _NEMBED + j, 16)] = a * bb
            return carry2

        lax.fori_loop(0, _CB, mul_b, 0)
        pltpu.sync_copy(buf, h_hbm.at[pl.ds(b0, _CB), :])
        return carry

    lax.fori_loop(0, _NCH, per_chunk, 0)


def _sc_gather_mul(cids, vids, cnt_p, val_p):
    mesh = plsc.VectorSubcoreMesh(core_axis_name="c", subcore_axis_name="s")
    f = pl.kernel(
        _sc_body,
        out_type=jax.ShapeDtypeStruct((_B, _D), jnp.float32),
        mesh=mesh,
        scratch_types=[
            pltpu.VMEM((_GR + 16,), jnp.int32),
            pltpu.VMEM((_GR + 16,), jnp.int32),
            pltpu.VMEM((_GR,), jnp.int32),
            pltpu.VMEM((_GR,), jnp.int32),
            pltpu.VMEM((_GR, 128), jnp.float32),
            pltpu.VMEM((_GR, 128), jnp.float32),
            pltpu.VMEM((_CB, _D), jnp.float32),
            pltpu.SemaphoreType.DMA,
        ],
    )
    return f(cids, vids, cnt_p, val_p)


def _tc_body(h_ref, w_ref, b_ref, o_ref):
    t = jnp.tanh(h_ref[...]).astype(jnp.bfloat16)
    o_ref[...] = (
        jnp.dot(t, w_ref[...], preferred_element_type=jnp.float32) + b_ref[...]
    )


def _tc_mlp(h, W, b):
    mb = 2048
    return pl.pallas_call(
        _tc_body,
        grid=(_B // mb,),
        in_specs=[
            pl.BlockSpec((mb, _D), lambda i: (i, 0)),
            pl.BlockSpec((_D, _NHID), lambda i: (0, 0)),
            pl.BlockSpec((1, _NHID), lambda i: (0, 0)),
        ],
        out_specs=pl.BlockSpec((mb, _NHID), lambda i: (i, 0)),
        out_shape=jax.ShapeDtypeStruct((_B, _NHID), jnp.float32),
    )(h, W.astype(jnp.bfloat16), b.reshape(1, _NHID))


@jax.jit
def kernel(ctx, cnt_table, val_table, W, b):
    cids = ctx[0::2].T.reshape(-1)
    vids = ctx[1::2].T.reshape(-1)
    h = _sc_gather_mul(cids, vids,
                       cnt_table.reshape(-1, 128), val_table.reshape(-1, 128))
    out = _tc_mlp(h, W, b)
    return out[None, :, :]


# speedup vs baseline: 1.1026x; 1.1026x over previous
import jax
import jax.numpy as jnp
from jax import lax
from jax.experimental import pallas as pl
from jax.experimental.pallas import tpu as pltpu
from jax.experimental.pallas import tpu_sc as plsc

_K = 26
_NEMBED = 32
_NHID = 128
_B = 16384
_D = _K * _NEMBED
_N = 1000000

_CHUNK = 16000
_QROWS = _CHUNK // 4
_NSLAB = 63 * _QROWS

_INFO = plsc.get_sparse_core_info()
_NC = _INFO.num_cores
_NS = _INFO.num_subcores
_NW = _NC * _NS
_CB = 8
_NCH = _B // (_NW * _CB)
_GR = _CB * _K


def _repack_body(x_ref, o_ref):
    x = x_ref[...]
    ys = [
        jnp.transpose(x[:, _QROWS * p:_QROWS * (p + 1)])
        for p in range(4)
    ]
    o_ref[...] = jnp.concatenate(ys, axis=1)


def _tc_repack(t_T):
    return pl.pallas_call(
        _repack_body,
        grid=(63,),
        in_specs=[pl.BlockSpec((32, _CHUNK), lambda i: (0, i))],
        out_specs=pl.BlockSpec((_QROWS, 128), lambda i: (i, 0)),
        out_shape=jax.ShapeDtypeStruct((_NSLAB, 128), jnp.float32),
    )(t_T)


def _sc_body(gc_hbm, oc_hbm, gv_hbm, ov_hbm, cnt_hbm, val_hbm, h_hbm,
             gidx_c, gidx_v, off_c, off_v, slab_c, slab_v, buf, sem):
    wid = lax.axis_index("s") * _NC + lax.axis_index("c")

    def per_chunk(c, carry):
        i0 = pl.multiple_of((wid * _NCH + c) * _GR, _GR)
        b0 = pl.multiple_of((wid * _NCH + c) * _CB, _CB)
        pltpu.sync_copy(gc_hbm.at[pl.ds(i0, _GR)], gidx_c)
        pltpu.sync_copy(gv_hbm.at[pl.ds(i0, _GR)], gidx_v)
        pltpu.sync_copy(oc_hbm.at[pl.ds(i0, _GR)], off_c.at[pl.ds(0, _GR)])
        pltpu.sync_copy(ov_hbm.at[pl.ds(i0, _GR)], off_v.at[pl.ds(0, _GR)])
        cpy_c = pltpu.async_copy(cnt_hbm.at[gidx_c], slab_c, sem)
        cpy_v = pltpu.async_copy(val_hbm.at[gidx_v], slab_v, sem)
        cpy_c.wait()
        cpy_v.wait()

        def mul_b(b, carry2):
            r0 = b * _K
            vc0 = off_c[pl.ds(r0, 16)]
            vc1 = off_c[pl.ds(r0 + 16, 16)]
            vv0 = off_v[pl.ds(r0, 16)]
            vv1 = off_v[pl.ds(r0 + 16, 16)]
            for k in range(_K):
                oc = pl.multiple_of(vc0[k] if k < 16 else vc1[k - 16], _NEMBED)
                ov = pl.multiple_of(vv0[k] if k < 16 else vv1[k - 16], _NEMBED)
                r = r0 + k
                for j in (0, 16):
                    a = slab_c[r, pl.ds(pl.multiple_of(oc + j, 16), 16)]
                    bb = slab_v[r, pl.ds(pl.multiple_of(ov + j, 16), 16)]
                    buf[b, pl.ds(k * _NEMBED + j, 16)] = a * bb
            return carry2

        lax.fori_loop(0, _CB, mul_b, 0)
        pltpu.sync_copy(buf, h_hbm.at[pl.ds(b0, _CB), :])
        return carry

    lax.fori_loop(0, _NCH, per_chunk, 0)


def _sc_gather_mul(gc, oc, gv, ov, cnt_p, val_p):
    mesh = plsc.VectorSubcoreMesh(core_axis_name="c", subcore_axis_name="s")
    f = pl.kernel(
        _sc_body,
        out_type=jax.ShapeDtypeStruct((_B, _D), jnp.float32),
        mesh=mesh,
        scratch_types=[
            pltpu.VMEM((_GR,), jnp.int32),
            pltpu.VMEM((_GR,), jnp.int32),
            pltpu.VMEM((_GR + 16,), jnp.int32),
            pltpu.VMEM((_GR + 16,), jnp.int32),
            pltpu.VMEM((_GR, 128), jnp.float32),
            pltpu.VMEM((_GR, 128), jnp.float32),
            pltpu.VMEM((_CB, _D), jnp.float32),
            pltpu.SemaphoreType.DMA,
        ],
    )
    return f(gc, oc, gv, ov, cnt_p, val_p)


def _tc_body(h_ref, w_ref, b_ref, o_ref):
    t = jnp.tanh(h_ref[...]).astype(jnp.bfloat16)
    o_ref[...] = (
        jnp.dot(t, w_ref[...], preferred_element_type=jnp.float32) + b_ref[...]
    )


def _tc_mlp(h, W, b):
    mb = 2048
    return pl.pallas_call(
        _tc_body,
        grid=(_B // mb,),
        in_specs=[
            pl.BlockSpec((mb, _D), lambda i: (i, 0)),
            pl.BlockSpec((_D, _NHID), lambda i: (0, 0)),
            pl.BlockSpec((1, _NHID), lambda i: (0, 0)),
        ],
        out_specs=pl.BlockSpec((mb, _NHID), lambda i: (i, 0)),
        out_shape=jax.ShapeDtypeStruct((_B, _NHID), jnp.float32),
    )(h, W.astype(jnp.bfloat16), b.reshape(1, _NHID))


@jax.jit
def kernel(ctx, cnt_table, val_table, W, b):
    cids = ctx[0::2].T.reshape(-1)
    vids = ctx[1::2].T.reshape(-1)
    gc = (cids // _CHUNK) * _QROWS + cids % _QROWS
    oc = ((cids % _CHUNK) // _QROWS) * _NEMBED
    gv = (vids // _CHUNK) * _QROWS + vids % _QROWS
    ov = ((vids % _CHUNK) // _QROWS) * _NEMBED
    h = _sc_gather_mul(gc, oc, gv, ov,
                       _tc_repack(cnt_table.T), _tc_repack(val_table.T))
    out = _tc_mlp(h, W, b)
    return out[None, :, :]


# speedup vs baseline: 2.3211x; 2.1051x over previous
import jax
import jax.numpy as jnp
from jax import lax
from jax.experimental import pallas as pl
from jax.experimental.pallas import tpu as pltpu
from jax.experimental.pallas import tpu_sc as plsc

_K = 26
_NEMBED = 32
_NHID = 128
_B = 16384
_D = _K * _NEMBED
_N = 1000000

_CHUNK = 25600
_QROWS = _CHUNK // 4
_NGRID = 40
_NSLAB = _NGRID * _QROWS

_INFO = plsc.get_sparse_core_info()
_NC = _INFO.num_cores
_NS = _INFO.num_subcores
_NW = _NC * _NS
_CB = 8
_NCH = _B // (_NW * _CB)
_GR = _CB * _K
_IW = 4 * _GR


def _repack_body(c_ref, v_ref, oc_ref, ov_ref):
    eye = jnp.eye(128, dtype=jnp.float32)
    for x_ref, o_ref in ((c_ref, oc_ref), (v_ref, ov_ref)):
        x = x_ref[...]
        xb = jnp.transpose(x.reshape(32, 4, _QROWS), (1, 0, 2)).reshape(
            128, _QROWS
        )
        o_ref[...] = jax.lax.dot_general(
            xb, eye,
            dimension_numbers=(((0,), (0,)), ((), ())),
            preferred_element_type=jnp.float32,
        )


def _tc_repack(cnt_T, val_T):
    return pl.pallas_call(
        _repack_body,
        grid=(_NGRID,),
        in_specs=[
            pl.BlockSpec((32, _CHUNK), lambda i: (0, i)),
            pl.BlockSpec((32, _CHUNK), lambda i: (0, i)),
        ],
        out_specs=[
            pl.BlockSpec((_QROWS, 128), lambda i: (i, 0)),
            pl.BlockSpec((_QROWS, 128), lambda i: (i, 0)),
        ],
        out_shape=[
            jax.ShapeDtypeStruct((_NSLAB, 128), jnp.float32),
            jax.ShapeDtypeStruct((_NSLAB, 128), jnp.float32),
        ],
    )(cnt_T, val_T)


def _sc_body(ids_hbm, cnt_hbm, val_hbm, h_hbm,
             idx_all, slab_c, slab_v, buf, sem0, sem1, semw):
    wid = lax.axis_index("s") * _NC + lax.axis_index("c")
    sems = (sem0, sem1)

    def stage_and_fire(c, p):
        q0 = pl.multiple_of((wid * _NCH + c) * _IW, _IW)
        pltpu.sync_copy(ids_hbm.at[pl.ds(q0, _IW)],
                        idx_all.at[pl.ds(p * _IW, _IW)])
        gi_c = idx_all.at[pl.ds(p * _IW, _GR)]
        gi_v = idx_all.at[pl.ds(p * _IW + 2 * _GR, _GR)]
        pltpu.async_copy(cnt_hbm.at[gi_c], slab_c.at[p], sems[p])
        pltpu.async_copy(val_hbm.at[gi_v], slab_v.at[p], sems[p])

    def consume(c, p):
        gi_c = idx_all.at[pl.ds(p * _IW, _GR)]
        gi_v = idx_all.at[pl.ds(p * _IW + 2 * _GR, _GR)]
        pltpu.make_async_copy(cnt_hbm.at[gi_c], slab_c.at[p], sems[p]).wait()
        pltpu.make_async_copy(val_hbm.at[gi_v], slab_v.at[p], sems[p]).wait()
        oc_base = p * _IW + _GR
        ov_base = p * _IW + 3 * _GR

        def mul_b(b, carry2):
            r0 = b * _K
            vc0 = idx_all[pl.ds(oc_base + r0, 16)]
            vc1 = idx_all[pl.ds(oc_base + r0 + 16, 16)]
            vv0 = idx_all[pl.ds(ov_base + r0, 16)]
            vv1 = idx_all[pl.ds(ov_base + r0 + 16, 16)]
            for k in range(_K):
                oc = pl.multiple_of(vc0[k] if k < 16 else vc1[k - 16], _NEMBED)
                ov = pl.multiple_of(vv0[k] if k < 16 else vv1[k - 16], _NEMBED)
                r = r0 + k
                for j in (0, 16):
                    a = slab_c[p, r, pl.ds(pl.multiple_of(oc + j, 16), 16)]
                    bb = slab_v[p, r, pl.ds(pl.multiple_of(ov + j, 16), 16)]
                    buf[p, b, pl.ds(k * _NEMBED + j, 16)] = a * bb
            return carry2

        lax.fori_loop(0, _CB, mul_b, 0)
        b0 = pl.multiple_of((wid * _NCH + c) * _CB, _CB)
        pltpu.sync_copy(buf.at[p], h_hbm.at[pl.ds(b0, _CB), :])

    stage_and_fire(0, 0)

    def outer(t, carry):
        c0 = 2 * t
        stage_and_fire(c0 + 1, 1)
        consume(c0, 0)

        @pl.when(c0 + 2 < _NCH)
        def _():
            stage_and_fire(c0 + 2, 0)

        consume(c0 + 1, 1)
        return carry

    lax.fori_loop(0, _NCH // 2, outer, 0)


def _sc_gather_mul(ids, cnt_p, val_p):
    mesh = plsc.VectorSubcoreMesh(core_axis_name="c", subcore_axis_name="s")
    f = pl.kernel(
        _sc_body,
        out_type=jax.ShapeDtypeStruct((_B, _D), jnp.float32),
        mesh=mesh,
        scratch_types=[
            pltpu.VMEM((2 * _IW,), jnp.int32),
            pltpu.VMEM((2, _GR, 128), jnp.float32),
            pltpu.VMEM((2, _GR, 128), jnp.float32),
            pltpu.VMEM((2, _CB, _D), jnp.float32),
            pltpu.SemaphoreType.DMA,
            pltpu.SemaphoreType.DMA,
            pltpu.SemaphoreType.DMA,
        ],
    )
    return f(ids, cnt_p, val_p)


def _tc_body(h_ref, w_ref, b_ref, o_ref):
    t = jnp.tanh(h_ref[...]).astype(jnp.bfloat16)
    o_ref[...] = (
        jnp.dot(t, w_ref[...], preferred_element_type=jnp.float32) + b_ref[...]
    )


def _tc_mlp(h, W, b):
    mb = 2048
    return pl.pallas_call(
        _tc_body,
        grid=(_B // mb,),
        in_specs=[
            pl.BlockSpec((mb, _D), lambda i: (i, 0)),
            pl.BlockSpec((_D, _NHID), lambda i: (0, 0)),
            pl.BlockSpec((1, _NHID), lambda i: (0, 0)),
        ],
        out_specs=pl.BlockSpec((mb, _NHID), lambda i: (i, 0)),
        out_shape=jax.ShapeDtypeStruct((_B, _NHID), jnp.float32),
    )(h, W.astype(jnp.bfloat16), b.reshape(1, _NHID))


@jax.jit
def kernel(ctx, cnt_table, val_table, W, b):
    cids = ctx[0::2].T.reshape(-1)
    vids = ctx[1::2].T.reshape(-1)
    gc = (cids // _CHUNK) * _QROWS + cids % _QROWS
    oc = ((cids % _CHUNK) // _QROWS) * _NEMBED
    gv = (vids // _CHUNK) * _QROWS + vids % _QROWS
    ov = ((vids % _CHUNK) // _QROWS) * _NEMBED
    ids = jnp.stack(
        [x.reshape(-1, _GR) for x in (gc, oc, gv, ov)], axis=1
    ).reshape(-1)
    cnt_p, val_p = _tc_repack(cnt_table.T, val_table.T)
    h = _sc_gather_mul(ids, cnt_p, val_p)
    out = _tc_mlp(h, W, b)
    return out[None, :, :]
